# baseline (speedup 1.0000x reference)
import functools

import jax
import jax.numpy as jnp
from jax import lax
from jax.experimental import pallas as pl
from jax.experimental.pallas import tpu as pltpu, tpu_sc as plsc

B, C, H, W = 64, 384, 28, 28
NC, NS, L = 2, 16, 16
NW = NC * NS
BPW = B // NW
K = 32
NCHUNK = C // K


def _body(x_hbm, perm_hbm, out_hbm,
          perm_v, buf0, buf1, buf2, gs0, gs1, gs2, ws0, ws1, ws2):
    wid = lax.axis_index("s") * NC + lax.axis_index("c")

    pltpu.sync_copy(perm_hbm, perm_v)

    bufs = (buf0, buf1, buf2)
    gsems = (gs0, gs1, gs2)
    wsems = (ws0, ws1, ws2)

    def copy_in(b, i, s):
        return pltpu.async_copy(
            x_hbm.at[b].at[perm_v.at[pl.ds(i * K, K)]], bufs[s], gsems[s])

    def copy_out(b, i, s):
        return pltpu.async_copy(bufs[s], out_hbm.at[b, pl.ds(i * K, K)],
                                wsems[s])

    total = BPW * NCHUNK
    chunks = [(wid * BPW + j // NCHUNK, j % NCHUNK) for j in range(total)]
    g = {}
    w = {}
    g[0] = copy_in(*chunks[0], 0)
    g[1] = copy_in(*chunks[1], 1)
    for j in range(total):
        s = j % 3
        if j + 2 < total:
            if j - 1 >= 0:
                w[j - 1].wait()
            g[j + 2] = copy_in(*chunks[j + 2], (j + 2) % 3)
        g[j].wait()
        w[j] = copy_out(*chunks[j], s)
    w[total - 3].wait()
    w[total - 2].wait()
    w[total - 1].wait()


@jax.jit
def _permute(x, perm):
    mesh = plsc.VectorSubcoreMesh(core_axis_name="c", subcore_axis_name="s")
    run = functools.partial(
        pl.kernel,
        mesh=mesh,
        compiler_params=pltpu.CompilerParams(use_tc_tiling_on_sc=False),
        out_type=jax.ShapeDtypeStruct((B, C, H * W), jnp.float32),
        scratch_types=[
            pltpu.VMEM((C,), jnp.int32),
            pltpu.VMEM((K, H * W), jnp.float32),
            pltpu.VMEM((K, H * W), jnp.float32),
            pltpu.VMEM((K, H * W), jnp.float32),
            pltpu.SemaphoreType.DMA,
            pltpu.SemaphoreType.DMA,
            pltpu.SemaphoreType.DMA,
            pltpu.SemaphoreType.DMA,
            pltpu.SemaphoreType.DMA,
            pltpu.SemaphoreType.DMA,
        ],
    )(_body)
    return run(x.reshape(B, C, H * W), perm)


def kernel(x, perm):
    y = _permute(x, perm).reshape(B, C, H, W)
    logdet = jnp.zeros((B,), dtype=x.dtype)
    return (y, logdet)
